# baseline (speedup 1.0000x reference)
import functools

import jax
import jax.numpy as jnp
from jax import lax
from jax.experimental import pallas as pl
from jax.experimental.pallas import tpu as pltpu
from jax.experimental.pallas import tpu_sc as plsc

_INFO = plsc.get_sparse_core_info()
_NC = _INFO.num_cores
_NS = _INFO.num_subcores
_NW = _NC * _NS
_L = _INFO.num_lanes


def _build(batch, seq, hidden):
    spw = seq // _NW
    q = 8
    ng = spw // q
    grows = batch * q
    rpw = batch * spw
    ring = 3
    nv = hidden // _L
    mesh = plsc.VectorSubcoreMesh(core_axis_name="c", subcore_axis_name="s")

    def body(tok_hbm, ids_hbm, pos_hbm, out_hbm,
             idx_v, pos_buf, tok_buf, pos_sem, gad_sem, out_sem):
        cid = lax.axis_index("c")
        sid = lax.axis_index("s")
        wid = sid * _NC + cid
        s_base = wid * spw

        for b in range(batch):
            pltpu.sync_copy(
                ids_hbm.at[pl.ds(b * seq + s_base, spw)],
                idx_v.at[pl.ds(b * spw, spw)])

        def _gather_descs(j):
            ts = lax.rem(j, ring)
            return [
                pltpu.make_async_copy(
                    tok_hbm.at[idx_v.at[pl.ds(b * spw + j * q, q)]],
                    tok_buf.at[ts, pl.ds(b * q, q)], gad_sem.at[ts])
                for b in range(batch)
            ]

        def _pos_desc(j):
            ps = lax.rem(j, ring)
            return pltpu.make_async_copy(
                pos_hbm.at[pl.ds(s_base + j * q, q)],
                pos_buf.at[ps], pos_sem.at[ps])

        def _scatter_descs(j):
            ts = lax.rem(j, ring)
            return [
                pltpu.make_async_copy(
                    tok_buf.at[ts, pl.ds(b * q, q)],
                    out_hbm.at[pl.ds(b * seq + s_base + j * q, q)],
                    out_sem.at[ts * batch + b])
                for b in range(batch)
            ]

        def gather(j):
            for d in _gather_descs(j):
                d.start()

        def pos_load(j):
            _pos_desc(j).start()

        def scatter(j):
            for d in _scatter_descs(j):
                d.start()

        gather(0)
        pos_load(0)
        pos_load(1)

        def group(j, _):
            ts = lax.rem(j, ring)

            @pl.when(j + 1 < ng)
            def _():
                @pl.when(j >= 2)
                def _():
                    for d in _scatter_descs(j - 2):
                        d.wait()
                gather(j + 1)

                @pl.when(j + 2 < ng)
                def _():
                    pos_load(j + 2)

            for d in _gather_descs(j):
                d.wait()
            _pos_desc(j).wait()

            def row(r, _):
                for k in range(nv):
                    sl = pl.ds(k * _L, _L)
                    p = pos_buf[ts, r, sl]
                    for b in range(batch):
                        tok_buf[ts, b * q + r, sl] = (
                            tok_buf[ts, b * q + r, sl] + p)
                return 0

            lax.fori_loop(0, q, row, 0)
            scatter(j)
            return 0

        lax.fori_loop(0, ng, group, 0)
        for j in (ng - 2, ng - 1):
            for d in _scatter_descs(j):
                d.wait()

    return pl.kernel(
        body,
        out_type=jax.ShapeDtypeStruct((batch * seq, hidden), jnp.float32),
        mesh=mesh,
        scratch_types=[
            pltpu.VMEM((rpw,), jnp.int32),
            pltpu.VMEM((ring, q, hidden), jnp.float32),
            pltpu.VMEM((ring, grows, hidden), jnp.float32),
            pltpu.SemaphoreType.DMA((ring,)),
            pltpu.SemaphoreType.DMA((ring,)),
            pltpu.SemaphoreType.DMA((ring * batch,)),
        ],
    )


@jax.jit
def kernel(input_ids, token_table, pos_table):
    b, seq = input_ids.shape
    hidden = token_table.shape[1]
    ids = input_ids.astype(jnp.int32).reshape(-1)
    out = _build(b, seq, hidden)(token_table, ids, pos_table)
    return out.reshape(b, seq, hidden)


# speedup vs baseline: 1.0001x; 1.0001x over previous
import functools

import jax
import jax.numpy as jnp
from jax import lax
from jax.experimental import pallas as pl
from jax.experimental.pallas import tpu as pltpu
from jax.experimental.pallas import tpu_sc as plsc

_INFO = plsc.get_sparse_core_info()
_NC = _INFO.num_cores
_NS = _INFO.num_subcores
_NW = _NC * _NS
_L = _INFO.num_lanes


def _build(batch, seq, hidden):
    spw = seq // _NW
    q = 8
    ng = spw // q
    grows = batch * q
    rpw = batch * spw
    ring = 3
    nv = hidden // _L
    kunroll = 16
    mesh = plsc.VectorSubcoreMesh(core_axis_name="c", subcore_axis_name="s")

    def body(tok_hbm, ids_hbm, pos_hbm, out_hbm,
             idx_b, idx_v, pos_buf, tok_buf, pos_sem, gad_sem, out_sem):
        cid = lax.axis_index("c")
        sid = lax.axis_index("s")
        wid = sid * _NC + cid
        s_base = wid * spw

        for b in range(batch):
            pltpu.sync_copy(
                ids_hbm.at[pl.ds(b * seq + s_base, spw)],
                idx_b.at[pl.ds(b * spw, spw)])
        low_half = lax.iota(jnp.int32, _L) < q
        for j in range(ng):
            for h in range(grows // _L):
                a = idx_b[pl.ds((2 * h) * spw + j * q, _L)]
                bvec = idx_b[pl.ds((2 * h + 1) * spw + j * q - q, _L)]
                idx_v[pl.ds(j * grows + h * _L, _L)] = jnp.where(
                    low_half, a, bvec)

        def _gather_desc(j):
            ts = lax.rem(j, ring)
            return pltpu.make_async_copy(
                tok_hbm.at[idx_v.at[pl.ds(j * grows, grows)]],
                tok_buf.at[ts], gad_sem.at[ts])

        def _pos_desc(j):
            ps = lax.rem(j, ring)
            return pltpu.make_async_copy(
                pos_hbm.at[pl.ds(s_base + j * q, q)],
                pos_buf.at[ps], pos_sem.at[ps])

        def _scatter_descs(j):
            ts = lax.rem(j, ring)
            return [
                pltpu.make_async_copy(
                    tok_buf.at[ts, pl.ds(b * q, q)],
                    out_hbm.at[pl.ds(b * seq + s_base + j * q, q)],
                    out_sem.at[ts * batch + b])
                for b in range(batch)
            ]

        def gather(j):
            _gather_desc(j).start()

        def pos_load(j):
            _pos_desc(j).start()

        def scatter(j):
            for d in _scatter_descs(j):
                d.start()

        gather(0)
        pos_load(0)
        pos_load(1)

        def group(j, _):
            ts = lax.rem(j, ring)

            @pl.when(j + 1 < ng)
            def _():
                @pl.when(j >= 2)
                def _():
                    for d in _scatter_descs(j - 2):
                        d.wait()
                gather(j + 1)

                @pl.when(j + 2 < ng)
                def _():
                    pos_load(j + 2)

            _gather_desc(j).wait()
            _pos_desc(j).wait()

            def row(r, _):
                for k in range(nv):
                    sl = pl.ds(k * _L, _L)
                    p = pos_buf[ts, r, sl]
                    for b in range(batch):
                        tok_buf[ts, b * q + r, sl] = (
                            tok_buf[ts, b * q + r, sl] + p)
                return 0

            lax.fori_loop(0, q, row, 0)
            scatter(j)
            return 0

        lax.fori_loop(0, ng, group, 0)
        for j in (ng - 2, ng - 1):
            for d in _scatter_descs(j):
                d.wait()

    return pl.kernel(
        body,
        out_type=jax.ShapeDtypeStruct((batch * seq, hidden), jnp.float32),
        mesh=mesh,
        scratch_types=[
            pltpu.VMEM((rpw,), jnp.int32),
            pltpu.VMEM((rpw,), jnp.int32),
            pltpu.VMEM((ring, q, hidden), jnp.float32),
            pltpu.VMEM((ring, grows, hidden), jnp.float32),
            pltpu.SemaphoreType.DMA((ring,)),
            pltpu.SemaphoreType.DMA((ring,)),
            pltpu.SemaphoreType.DMA((ring * batch,)),
        ],
    )


@jax.jit
def kernel(input_ids, token_table, pos_table):
    b, seq = input_ids.shape
    hidden = token_table.shape[1]
    ids = input_ids.astype(jnp.int32).reshape(-1)
    out = _build(b, seq, hidden)(token_table, ids, pos_table)
    return out.reshape(b, seq, hidden)
